# Initial kernel scaffold; baseline (speedup 1.0000x reference)
import jax
import jax.numpy as jnp
from jax.experimental import pallas as pl


def kernel(x, pos_embed):
    raise NotImplementedError("write your pallas kernel here")


# speedup vs baseline: 1.6651x; 1.6651x over previous
import jax
import jax.numpy as jnp
from jax.experimental import pallas as pl


def _pe_add_kernel(x_ref, pe_ref, o_ref):
    o_ref[...] = x_ref[...] + pe_ref[...]


def kernel(x, pos_embed):
    B, S, D = x.shape
    BS = 512
    return pl.pallas_call(
        _pe_add_kernel,
        grid=(S // BS, B),
        in_specs=[
            pl.BlockSpec((1, BS, D), lambda s, b: (b, s, 0)),
            pl.BlockSpec((BS, D), lambda s, b: (s, 0)),
        ],
        out_specs=pl.BlockSpec((1, BS, D), lambda s, b: (b, s, 0)),
        out_shape=jax.ShapeDtypeStruct(x.shape, x.dtype),
    )(x, pos_embed)


# speedup vs baseline: 1.8510x; 1.1116x over previous
import jax
import jax.numpy as jnp
from jax.experimental import pallas as pl


def _pe_add_kernel(x_ref, pe_ref, o_ref):
    o_ref[...] = x_ref[...] + pe_ref[...]


def kernel(x, pos_embed):
    B, S, D = x.shape
    BS = 1024
    return pl.pallas_call(
        _pe_add_kernel,
        grid=(S // BS, B),
        in_specs=[
            pl.BlockSpec((1, BS, D), lambda s, b: (b, s, 0)),
            pl.BlockSpec((BS, D), lambda s, b: (s, 0)),
        ],
        out_specs=pl.BlockSpec((1, BS, D), lambda s, b: (b, s, 0)),
        out_shape=jax.ShapeDtypeStruct(x.shape, x.dtype),
    )(x, pos_embed)


# speedup vs baseline: 1.9691x; 1.0638x over previous
import jax
import jax.numpy as jnp
from jax.experimental import pallas as pl


def _pe_add_kernel(x_ref, pe_ref, o_ref):
    o_ref[...] = x_ref[...] + pe_ref[...]


def kernel(x, pos_embed):
    B, S, D = x.shape
    BS = 2048
    return pl.pallas_call(
        _pe_add_kernel,
        grid=(S // BS, B),
        in_specs=[
            pl.BlockSpec((1, BS, D), lambda s, b: (b, s, 0)),
            pl.BlockSpec((BS, D), lambda s, b: (s, 0)),
        ],
        out_specs=pl.BlockSpec((1, BS, D), lambda s, b: (b, s, 0)),
        out_shape=jax.ShapeDtypeStruct(x.shape, x.dtype),
    )(x, pos_embed)
